# baseline (speedup 1.0000x reference)
import functools

import jax
import jax.numpy as jnp
from jax import lax
from jax.experimental import pallas as pl
from jax.experimental.pallas import tpu as pltpu
from jax.experimental.pallas import tpu_sc as plsc

NUM_HEADS = 26
N_PER_HEAD = 100000
D = 32
BATCH = 16384
TOTAL = BATCH * NUM_HEADS
NUM_ROWS = NUM_HEADS * N_PER_HEAD
NUM_WORKERS = 32
COL_GROUPS = NUM_ROWS // 128
TAIL_BASE = COL_GROUPS * 128
SCRATCH_ROWS = NUM_ROWS
GROUPS_PER_W = (COL_GROUPS + NUM_WORKERS - 1) // NUM_WORKERS
UNITS = NUM_HEADS * (BATCH // 128)
UNITS_PER_W = UNITS // NUM_WORKERS


def _detile_body(ids_hbm, tab_hbm, scr_hbm, fid_hbm, in_v, out_v, id_v, sem):
    wid = lax.axis_index("s") * 2 + lax.axis_index("c")
    lane = lax.iota(jnp.int32, 16)

    def unit(u, carry):
        gu = wid * UNITS_PER_W + u
        h = gu // 128
        s = gu % 128
        pltpu.sync_copy(ids_hbm.at[h, pl.ds(s * 128, 128)], id_v)
        off = h * N_PER_HEAD

        def addv(j, c):
            id_v[pl.ds(j * 16, 16)] = id_v[pl.ds(j * 16, 16)] + off
            return c

        lax.fori_loop(0, 8, addv, 0)
        pltpu.sync_copy(id_v, fid_hbm.at[pl.ds(gu * 128, 128)])
        return carry

    lax.fori_loop(0, UNITS_PER_W, unit, 0)

    def group(k, carry):
        g = wid + k * NUM_WORKERS

        @pl.when(g < COL_GROUPS)
        def _():
            base = g * 128
            pltpu.sync_copy(tab_hbm.at[:, pl.ds(base, 128)], in_v)

            def tvec(k2, c):
                col = k2 // 2
                d0 = (k2 % 2) * 16
                dv = lane + d0
                cv = jnp.full((16,), 0, jnp.int32) + col
                out_v[col, pl.ds(d0, 16)] = plsc.load_gather(in_v, [dv, cv])
                return c

            lax.fori_loop(0, 256, tvec, 0)
            pltpu.sync_copy(out_v, scr_hbm.at[pl.ds(base, 128), :])

        return carry

    lax.fori_loop(0, GROUPS_PER_W, group, 0)


_detile = functools.partial(
    pl.kernel,
    out_type=(
        jax.ShapeDtypeStruct((SCRATCH_ROWS, D), jnp.float32),
        jax.ShapeDtypeStruct((TOTAL,), jnp.int32),
    ),
    scratch_types=[
        pltpu.VMEM((D, 128), jnp.float32),
        pltpu.VMEM((128, D), jnp.float32),
        pltpu.VMEM((128,), jnp.int32),
        pltpu.SemaphoreType.DMA,
    ],
    mesh=plsc.VectorSubcoreMesh(core_axis_name="c", subcore_axis_name="s"),
    compiler_params=pltpu.CompilerParams(needs_layout_passes=False),
)(_detile_body)


def _gather_body(fid_hbm, scr_hbm, tail_hbm, out_hbm, id_v, rm_v, rows_v, t_v, sem):
    wid = lax.axis_index("s") * 2 + lax.axis_index("c")
    lane = lax.iota(jnp.int32, 16)
    pltpu.sync_copy(tail_hbm, rows_v.at[pl.ds(128, 64)])

    def unit(u, carry):
        gu = wid * UNITS_PER_W + u
        h = gu // 128
        s = gu % 128
        pltpu.sync_copy(fid_hbm.at[pl.ds(gu * 128, 128)], id_v)
        pltpu.async_copy(scr_hbm.at[id_v], rows_v.at[pl.ds(0, 128)], sem).wait()

        def rmap(j, c):
            iv = id_v[pl.ds(j * 16, 16)]
            rm_v[pl.ds(j * 16, 16)] = jnp.where(
                iv >= TAIL_BASE, iv - (TAIL_BASE - 128), lane + j * 16
            )
            return c

        lax.fori_loop(0, 8, rmap, 0)

        def tvec(k2, c):
            d = k2 // 8
            c0 = (k2 % 8) * 16
            q = k2 // 64
            f = (k2 // 8) % 8
            dv = jnp.full((16,), 0, jnp.int32) + d
            rv = rm_v[pl.ds(c0, 16)]
            t_v[q, f, pl.ds(c0, 16)] = plsc.load_gather(rows_v, [rv, dv])
            return c

        lax.fori_loop(0, 256, tvec, 0)
        for q in range(4):
            pltpu.sync_copy(t_v.at[q], out_hbm.at[h, q, s])
        return carry

    lax.fori_loop(0, UNITS_PER_W, unit, 0)


_gather = functools.partial(
    pl.kernel,
    out_type=jax.ShapeDtypeStruct((NUM_HEADS, 4, 128, 8, 128), jnp.float32),
    scratch_types=[
        pltpu.VMEM((128,), jnp.int32),
        pltpu.VMEM((128,), jnp.int32),
        pltpu.VMEM((192, D), jnp.float32),
        pltpu.VMEM((4, 8, 128), jnp.float32),
        pltpu.SemaphoreType.DMA,
    ],
    mesh=plsc.VectorSubcoreMesh(core_axis_name="c", subcore_axis_name="s"),
    compiler_params=pltpu.CompilerParams(
        use_tc_tiling_on_sc=False, needs_layout_passes=False
    ),
)(_gather_body)


def kernel(input_ids, table):
    ids_t = input_ids.T
    tab_t = table.T
    tail = lax.slice(table, (TAIL_BASE, 0), (NUM_ROWS, D))
    scr, fid = _detile(ids_t, tab_t)
    out5 = _gather(fid, scr, tail)
    out3 = out5.transpose(0, 1, 3, 2, 4).reshape(NUM_HEADS, D, BATCH)
    return out3.transpose(2, 0, 1)


# speedup vs baseline: 2.0877x; 2.0877x over previous
import functools

import jax
import jax.numpy as jnp
from jax import lax
from jax.experimental import pallas as pl
from jax.experimental.pallas import tpu as pltpu
from jax.experimental.pallas import tpu_sc as plsc

NUM_HEADS = 26
N_PER_HEAD = 100000
D = 32
BATCH = 16384
TOTAL = BATCH * NUM_HEADS
NUM_ROWS = NUM_HEADS * N_PER_HEAD
NUM_WORKERS = 32
COL_GROUPS = NUM_ROWS // 128
TAIL_BASE = COL_GROUPS * 128
GROUPS_PER_W = (COL_GROUPS + NUM_WORKERS - 1) // NUM_WORKERS
UNITS_PER_W = (NUM_HEADS * (BATCH // 128)) // NUM_WORKERS
IDS_PER_W = TOTAL // NUM_WORKERS


def _detile_body(ids_hbm, tab_hbm, scr_hbm, fid_hbm,
                 in0, in1, ou0, ou1, idb,
                 is0, is1, os0, os1):
    wid = lax.axis_index("s") * 2 + lax.axis_index("c")
    lane = lax.iota(jnp.int32, 16)
    dv0 = lane
    dv1 = lane + 16

    q0 = wid * IDS_PER_W

    def idchunk(j, carry):
        q = q0 + j * 512
        pltpu.sync_copy(
            ids_hbm.at[q // BATCH, pl.ds(q % BATCH, 512)],
            idb.at[pl.ds(j * 512, 512)],
        )
        return carry

    lax.fori_loop(0, IDS_PER_W // 512, idchunk, 0)

    @plsc.parallel_loop(0, IDS_PER_W // 16, unroll=8)
    def _(i):
        off = ((q0 + i * 16) // BATCH) * N_PER_HEAD
        idb[pl.ds(i * 16, 16)] = idb[pl.ds(i * 16, 16)] + off

    pltpu.sync_copy(idb, fid_hbm.at[pl.ds(q0, IDS_PER_W)])

    n_mine = (COL_GROUPS - wid + NUM_WORKERS - 1) // NUM_WORKERS
    ins = (in0, in1)
    ous = (ou0, ou1)
    isems = (is0, is1)
    osems = (os0, os1)

    def issue_in(k, b):
        g = wid + k * NUM_WORKERS
        pltpu.async_copy(tab_hbm.at[:, pl.ds(g * 128, 128)], ins[b], isems[b])

    def wait_in(b):
        pltpu.make_async_copy(
            tab_hbm.at[:, pl.ds(0, 128)], ins[b], isems[b]
        ).wait()

    def issue_out(k, b):
        g = wid + k * NUM_WORKERS
        pltpu.async_copy(ous[b], scr_hbm.at[pl.ds(g * 128, 128), :], osems[b])

    def wait_out(b):
        pltpu.make_async_copy(
            ous[b], scr_hbm.at[pl.ds(0, 128), :], osems[b]
        ).wait()

    def transpose(b):
        src = ins[b]
        dst = ous[b]

        @plsc.parallel_loop(0, 128, unroll=8)
        def _(c):
            cv = jnp.full((16,), 0, jnp.int32) + c
            dst[c, pl.ds(0, 16)] = plsc.load_gather(src, [dv0, cv])
            dst[c, pl.ds(16, 16)] = plsc.load_gather(src, [dv1, cv])

    @pl.when(n_mine > 0)
    def _():
        issue_in(0, 0)

    def outer(k2, carry):
        for b in (0, 1):
            k = k2 * 2 + b

            @pl.when(k + 1 < n_mine)
            def _():
                issue_in(k + 1, 1 - b)

            @pl.when(k < n_mine)
            def _():
                wait_in(b)

                @pl.when(k >= 2)
                def _():
                    wait_out(b)

                transpose(b)
                issue_out(k, b)

        return carry

    lax.fori_loop(0, (GROUPS_PER_W + 1) // 2, outer, 0)

    @pl.when(n_mine >= 1)
    def _():
        wait_out(0)

    @pl.when(n_mine >= 2)
    def _():
        wait_out(1)


_detile = functools.partial(
    pl.kernel,
    out_type=(
        jax.ShapeDtypeStruct((NUM_ROWS, D), jnp.float32),
        jax.ShapeDtypeStruct((TOTAL,), jnp.int32),
    ),
    scratch_types=[
        pltpu.VMEM((D, 128), jnp.float32),
        pltpu.VMEM((D, 128), jnp.float32),
        pltpu.VMEM((128, D), jnp.float32),
        pltpu.VMEM((128, D), jnp.float32),
        pltpu.VMEM((IDS_PER_W,), jnp.int32),
        pltpu.SemaphoreType.DMA,
        pltpu.SemaphoreType.DMA,
        pltpu.SemaphoreType.DMA,
        pltpu.SemaphoreType.DMA,
    ],
    mesh=plsc.VectorSubcoreMesh(core_axis_name="c", subcore_axis_name="s"),
    compiler_params=pltpu.CompilerParams(needs_layout_passes=False),
)(_detile_body)


def _gather_body(fid_hbm, scr_hbm, tail_hbm, out_hbm,
                 id0, id1, rm_v, r0, r1, t0, t1,
                 gs0, gs1, js0, js1, ws0, ws1):
    wid = lax.axis_index("s") * 2 + lax.axis_index("c")
    lane = lax.iota(jnp.int32, 16)
    pltpu.sync_copy(tail_hbm, r0.at[pl.ds(128, 64)])
    pltpu.sync_copy(tail_hbm, r1.at[pl.ds(128, 64)])

    ids = (id0, id1)
    rs = (r0, r1)
    ts = (t0, t1)
    gsems = (gs0, gs1)
    jsems = (js0, js1)
    wsems = (ws0, ws1)
    u0 = wid * UNITS_PER_W

    def issue_ids(k, b):
        pltpu.async_copy(
            fid_hbm.at[pl.ds((u0 + k) * 128, 128)], ids[b], jsems[b]
        )

    def wait_ids(b):
        pltpu.make_async_copy(
            fid_hbm.at[pl.ds(0, 128)], ids[b], jsems[b]
        ).wait()

    def issue_gather(b):
        pltpu.async_copy(scr_hbm.at[ids[b]], rs[b].at[pl.ds(0, 128)], gsems[b])

    def wait_gather(b):
        pltpu.make_async_copy(
            scr_hbm.at[ids[b]], rs[b].at[pl.ds(0, 128)], gsems[b]
        ).wait()

    def issue_out(k, b):
        gu = u0 + k
        h = gu // 128
        s = gu % 128
        for q in range(4):
            pltpu.async_copy(ts[b].at[q], out_hbm.at[h, q, s], wsems[b])

    def wait_out(b):
        for q in range(4):
            pltpu.make_async_copy(
                ts[b].at[q], out_hbm.at[0, q, 0], wsems[b]
            ).wait()

    def transpose(b):
        src = rs[b]
        dst = ts[b]

        @plsc.parallel_loop(0, 8, unroll=8)
        def _(c8):
            c0 = c8 * 16
            rv = rm_v[pl.ds(c0, 16)]
            for d in range(32):
                dv = jnp.full((16,), 0, jnp.int32) + d
                dst[d // 8, d % 8, pl.ds(c0, 16)] = plsc.load_gather(
                    src, [rv, dv]
                )

    pltpu.sync_copy(fid_hbm.at[pl.ds(u0 * 128, 128)], id0)
    issue_gather(0)
    issue_ids(1, 1)

    def outer(k2, carry):
        for b in (0, 1):
            k = k2 * 2 + b
            wait_gather(b)

            @pl.when(k + 1 < UNITS_PER_W)
            def _():
                wait_ids(1 - b)
                issue_gather(1 - b)

            @plsc.parallel_loop(0, 8, unroll=8)
            def _(j):
                iv = ids[b][pl.ds(j * 16, 16)]
                rm_v[pl.ds(j * 16, 16)] = jnp.where(
                    iv >= TAIL_BASE, iv - (TAIL_BASE - 128), lane + j * 16
                )

            @pl.when(k + 2 < UNITS_PER_W)
            def _():
                issue_ids(k + 2, b)

            @pl.when(k >= 2)
            def _():
                wait_out(b)

            transpose(b)
            issue_out(k, b)

        return carry

    lax.fori_loop(0, UNITS_PER_W // 2, outer, 0)
    wait_out(0)
    wait_out(1)


_gather = functools.partial(
    pl.kernel,
    out_type=jax.ShapeDtypeStruct((NUM_HEADS, 4, 128, 8, 128), jnp.float32),
    scratch_types=[
        pltpu.VMEM((128,), jnp.int32),
        pltpu.VMEM((128,), jnp.int32),
        pltpu.VMEM((128,), jnp.int32),
        pltpu.VMEM((192, D), jnp.float32),
        pltpu.VMEM((192, D), jnp.float32),
        pltpu.VMEM((4, 8, 128), jnp.float32),
        pltpu.VMEM((4, 8, 128), jnp.float32),
        pltpu.SemaphoreType.DMA,
        pltpu.SemaphoreType.DMA,
        pltpu.SemaphoreType.DMA,
        pltpu.SemaphoreType.DMA,
        pltpu.SemaphoreType.DMA,
        pltpu.SemaphoreType.DMA,
    ],
    mesh=plsc.VectorSubcoreMesh(core_axis_name="c", subcore_axis_name="s"),
    compiler_params=pltpu.CompilerParams(
        use_tc_tiling_on_sc=False, needs_layout_passes=False
    ),
)(_gather_body)


def kernel(input_ids, table):
    ids_t = input_ids.T
    tab_t = table.T
    tail = lax.slice(table, (TAIL_BASE, 0), (NUM_ROWS, D))
    scr, fid = _detile(ids_t, tab_t)
    out5 = _gather(fid, scr, tail)
    out3 = out5.transpose(0, 1, 3, 2, 4).reshape(NUM_HEADS, D, BATCH)
    return out3.transpose(2, 0, 1)
